# baseline (speedup 1.0000x reference)
import functools

import jax
import jax.numpy as jnp
from jax import lax
from jax.experimental import pallas as pl
from jax.experimental.pallas import tpu as pltpu
from jax.experimental.pallas import tpu_sc as plsc

K = 8192
D = 256
BETA = 0.25

N_BLK = 4096
K_WIN = 2048
LANES = 128


def _argmin_body(z_ref, e_ref, zsq_ref, esq_ref, idx_ref, dmin_ref):
    z = z_ref[...]
    zsq = zsq_ref[...]
    lane = lax.broadcasted_iota(jnp.int32, (N_BLK, LANES), 1)

    r_v = r_i = r_f = None
    for w in range(K // K_WIN):
        mm = lax.dot_general(z, e_ref[w * K_WIN:(w + 1) * K_WIN, :],
                             (((1,), (1,)), ((), ())),
                             preferred_element_type=jnp.float32)
        mm2 = 2.0 * mm
        d = (zsq + esq_ref[:, w * K_WIN:(w + 1) * K_WIN]) - mm2
        bv = d[:, 0:LANES]
        bj = jnp.zeros((N_BLK, LANES), dtype=jnp.int32)
        for j in range(1, K_WIN // LANES):
            dj = d[:, j * LANES:(j + 1) * LANES]
            lt = dj < bv
            bv = jnp.where(lt, dj, bv)
            bj = jnp.where(lt, j, bj)
        wmin = jnp.min(bv, axis=1, keepdims=True)
        col = bj * LANES + lane + (w * K_WIN)
        wcol = jnp.min(jnp.where(bv == wmin, col, jnp.int32(2**31 - 1)),
                       axis=1, keepdims=True)
        if w == 0:
            r_i, r_f = wcol, wmin
        else:
            take = wmin < r_v
            r_i = jnp.where(take, wcol, r_i)
            r_f = jnp.where(take, wmin, r_f)
        r_v = r_f.astype(jnp.bfloat16).astype(jnp.float32)
    idx_ref[...] = r_i
    dmin_ref[...] = r_f


def _argmin_call(flat, et, zsq, esq):
    n = flat.shape[0]
    grid = (n // N_BLK,)
    return pl.pallas_call(
        _argmin_body,
        grid=grid,
        in_specs=[
            pl.BlockSpec((N_BLK, D), lambda i: (i, 0)),
            pl.BlockSpec((K, D), lambda i: (0, 0)),
            pl.BlockSpec((N_BLK, 1), lambda i: (i, 0)),
            pl.BlockSpec((1, K), lambda i: (0, 0)),
        ],
        out_specs=[
            pl.BlockSpec((N_BLK, 1), lambda i: (i, 0)),
            pl.BlockSpec((N_BLK, 1), lambda i: (i, 0)),
        ],
        out_shape=[
            jax.ShapeDtypeStruct((n, 1), jnp.int32),
            jax.ShapeDtypeStruct((n, 1), jnp.float32),
        ],
    )(flat, et, zsq, esq)


def _make_sc_gather(n):
    info = plsc.get_sparse_core_info()
    nw = info.num_cores * info.num_subcores
    b_per_w = n // nw
    mesh = plsc.VectorSubcoreMesh(core_axis_name="c", subcore_axis_name="s")

    @functools.partial(
        pl.kernel, mesh=mesh,
        out_type=jax.ShapeDtypeStruct((n, D), jnp.float32),
        scratch_types=[
            pltpu.VMEM((b_per_w,), jnp.int32),
            pltpu.VMEM((b_per_w, D), jnp.float32),
            pltpu.SemaphoreType.DMA,
        ],
    )
    def gather(table_hbm, idx_hbm, out_hbm, idx_v, rows_v, sem):
        wid = lax.axis_index("s") * info.num_cores + lax.axis_index("c")
        base = wid * b_per_w
        pltpu.sync_copy(idx_hbm.at[pl.ds(base, b_per_w)], idx_v)
        pltpu.async_copy(table_hbm.at[idx_v], rows_v, sem).wait()
        pltpu.sync_copy(rows_v, out_hbm.at[pl.ds(base, b_per_w)])

    return gather


def kernel(latents, embedding_weight):
    shape = latents.shape
    flat = latents.reshape(-1, shape[-1])
    n = flat.shape[0]
    zsq = jnp.sum(flat ** 2, axis=1, keepdims=True)
    esq = jnp.sum(embedding_weight ** 2, axis=1).reshape(1, K)

    idx, dmin = _argmin_call(flat, embedding_weight, zsq, esq)

    quantized = _make_sc_gather(n)(embedding_weight, idx.reshape(n))

    v = jnp.sum(dmin) / (n * D)
    vq_loss = v * BETA + v
    return (quantized.reshape(shape), vq_loss)


# speedup vs baseline: 1.1923x; 1.1923x over previous
import functools

import jax
import jax.numpy as jnp
from jax import lax
from jax.experimental import pallas as pl
from jax.experimental.pallas import tpu as pltpu
from jax.experimental.pallas import tpu_sc as plsc

K = 8192
D = 256
BETA = 0.25

N_BLK = 2048
K_WIN = 2048
LANES = 128


def _argmin_body(z_ref, e_ref, zsq_ref, esq_ref, idx_ref, dmin_ref):
    z = z_ref[...]
    zsq = zsq_ref[...]
    lane = lax.broadcasted_iota(jnp.int32, (N_BLK, LANES), 1)

    r_v = r_i = r_f = None
    for w in range(K // K_WIN):
        mm = lax.dot_general(z, e_ref[w * K_WIN:(w + 1) * K_WIN, :],
                             (((1,), (1,)), ((), ())),
                             preferred_element_type=jnp.float32)
        mm2 = 2.0 * mm
        d = (zsq + esq_ref[:, w * K_WIN:(w + 1) * K_WIN]) - mm2
        bv = d[:, 0:LANES]
        bj = jnp.zeros((N_BLK, LANES), dtype=jnp.int32)
        for j in range(1, K_WIN // LANES):
            dj = d[:, j * LANES:(j + 1) * LANES]
            lt = dj < bv
            bv = jnp.where(lt, dj, bv)
            bj = jnp.where(lt, j, bj)
        wmin = jnp.min(bv, axis=1, keepdims=True)
        col = bj * LANES + lane + (w * K_WIN)
        wcol = jnp.min(jnp.where(bv == wmin, col, jnp.int32(2**31 - 1)),
                       axis=1, keepdims=True)
        if w == 0:
            r_i, r_f = wcol, wmin
        else:
            take = wmin < r_v
            r_i = jnp.where(take, wcol, r_i)
            r_f = jnp.where(take, wmin, r_f)
        r_v = r_f.astype(jnp.bfloat16).astype(jnp.float32)
    idx_ref[...] = r_i
    dmin_ref[...] = r_f


def _argmin_call(flat, et, zsq, esq):
    n = flat.shape[0]
    grid = (n // N_BLK,)
    return pl.pallas_call(
        _argmin_body,
        grid=grid,
        in_specs=[
            pl.BlockSpec((N_BLK, D), lambda i: (i, 0)),
            pl.BlockSpec((K, D), lambda i: (0, 0)),
            pl.BlockSpec((N_BLK, 1), lambda i: (i, 0)),
            pl.BlockSpec((1, K), lambda i: (0, 0)),
        ],
        out_specs=[
            pl.BlockSpec((N_BLK, 1), lambda i: (i, 0)),
            pl.BlockSpec((N_BLK, 1), lambda i: (i, 0)),
        ],
        out_shape=[
            jax.ShapeDtypeStruct((n, 1), jnp.int32),
            jax.ShapeDtypeStruct((n, 1), jnp.float32),
        ],
    )(flat, et, zsq, esq)


def _make_sc_gather(n):
    info = plsc.get_sparse_core_info()
    nw = info.num_cores * info.num_subcores
    b_per_w = n // nw
    mesh = plsc.VectorSubcoreMesh(core_axis_name="c", subcore_axis_name="s")

    @functools.partial(
        pl.kernel, mesh=mesh,
        out_type=jax.ShapeDtypeStruct((n, D), jnp.float32),
        scratch_types=[
            pltpu.VMEM((b_per_w,), jnp.int32),
            pltpu.VMEM((b_per_w, D), jnp.float32),
            pltpu.SemaphoreType.DMA,
        ],
    )
    def gather(table_hbm, idx_hbm, out_hbm, idx_v, rows_v, sem):
        wid = lax.axis_index("s") * info.num_cores + lax.axis_index("c")
        base = wid * b_per_w
        pltpu.sync_copy(idx_hbm.at[pl.ds(base, b_per_w)], idx_v)
        pltpu.async_copy(table_hbm.at[idx_v], rows_v, sem).wait()
        pltpu.sync_copy(rows_v, out_hbm.at[pl.ds(base, b_per_w)])

    return gather


def kernel(latents, embedding_weight):
    shape = latents.shape
    flat = latents.reshape(-1, shape[-1])
    n = flat.shape[0]
    zsq = jnp.sum(flat ** 2, axis=1, keepdims=True)
    esq = jnp.sum(embedding_weight ** 2, axis=1).reshape(1, K)

    idx, dmin = _argmin_call(flat, embedding_weight, zsq, esq)

    quantized = _make_sc_gather(n)(embedding_weight, idx.reshape(n))

    v = jnp.sum(dmin) / (n * D)
    vq_loss = v * BETA + v
    return (quantized.reshape(shape), vq_loss)
